# Initial kernel scaffold; baseline (speedup 1.0000x reference)
import jax
import jax.numpy as jnp
from jax.experimental import pallas as pl


def kernel(x_user, x_course, W1_rates_l, b1_rates, W1_rates_r, W1_rev_l, b1_rev, W1_rev_r, W2_rates_l, b2_rates, W2_rates_r, W2_rev_l, b2_rev, W2_rev_r, Wd1, bd1, Wd2, bd2, ei_u2c, ei_c2u, edge_label_index):
    raise NotImplementedError("write your pallas kernel here")


# speedup vs baseline: 1.5315x; 1.5315x over previous
import functools

import jax
import jax.numpy as jnp
from jax import lax
from jax.experimental import pallas as pl
from jax.experimental.pallas import tpu as pltpu
from jax.experimental.pallas import tpu_sc as plsc

N = 10000
D = 256
E = 160000
NL = 50000

NC, NS = 2, 16
E_PER_TILE = 10240
E_PAD = NS * E_PER_TILE
CH = 128
N_CHUNKS = E_PER_TILE // CH
ACC_ROWS = 10240
ZR = ACC_ROWS // NS
NZB = ZR // CH
TRASH = N
W = 128

NLP = 53248
L_PER_TILE = NLP // (NC * NS)
LCH = 128
L_CHUNKS = L_PER_TILE // LCH

BM = 2000
_DN = (((1,), (1,)), ((), ()))


def _dot(x, w):
    return lax.dot_general(x, w, _DN, preferred_element_type=jnp.float32)


_GDN = lax.GatherDimensionNumbers(offset_dims=(), collapsed_slice_dims=(0,),
                                  start_index_map=(0,))


def _shuffle(v, idx):
    return lax.gather(v, idx[:, None], _GDN, (1,),
                      mode=lax.GatherScatterMode.PROMISE_IN_BOUNDS)


def _t1_body(xu_ref, xc_ref, w1l_ref, w1r_ref, wr1l_ref, wr1r_ref,
             b1_ref, br1_ref, g_ref, bc_ref, bu_ref):
    xu = xu_ref[...]
    xc = xc_ref[...]
    g1u = _dot(xu, w1l_ref[...])
    g1c = _dot(xc, wr1l_ref[...])
    g_ref[0] = g1u[:, :128]
    g_ref[1] = g1u[:, 128:]
    g_ref[2] = g1c[:, :128]
    g_ref[3] = g1c[:, 128:]
    bc_ref[...] = _dot(xc, w1r_ref[...]) + b1_ref[...]
    bu_ref[...] = _dot(xu, wr1r_ref[...]) + br1_ref[...]


def _t2_body(agg_ref, bc_ref, bu_ref, w2l_ref, w2r_ref,
             wr2l_ref, wr2r_ref, b2_ref, br2_ref,
             g2_ref, b2c_ref, b2u_ref, rdeg_ref):
    aggc = jnp.concatenate([agg_ref[0, 0], agg_ref[0, 1]], axis=1)
    aggu = jnp.concatenate([agg_ref[1, 0], agg_ref[1, 1]], axis=1)
    rdegc = 1.0 / jnp.maximum(agg_ref[0, 2][:, 0:1], 1.0)
    rdegu = 1.0 / jnp.maximum(agg_ref[1, 2][:, 0:1], 1.0)
    c1 = jnp.maximum(aggc * rdegc + bc_ref[...], 0.0)
    u1 = jnp.maximum(aggu * rdegu + bu_ref[...], 0.0)
    g2u = _dot(u1, w2l_ref[...])
    g2c = _dot(c1, wr2l_ref[...])
    g2_ref[0] = g2u[:, :128]
    g2_ref[1] = g2u[:, 128:]
    g2_ref[2] = g2c[:, :128]
    g2_ref[3] = g2c[:, 128:]
    b2c_ref[...] = _dot(c1, w2r_ref[...]) + b2_ref[...]
    b2u_ref[...] = _dot(u1, wr2r_ref[...]) + br2_ref[...]
    bm = rdegc.shape[0]
    rdeg_ref[0] = jnp.broadcast_to(rdegc, (bm, 16))
    rdeg_ref[1] = jnp.broadcast_to(rdegu, (bm, 16))


def _t3_body(agg_ref, rdeg_ref, b2c_ref, b2u_ref, wd1l_ref, wd1r_ref,
             bd1_ref, p_ref, q_ref):
    aggc = jnp.concatenate([agg_ref[0, 0], agg_ref[0, 1]], axis=1)
    aggu = jnp.concatenate([agg_ref[1, 0], agg_ref[1, 1]], axis=1)
    c2 = aggc * rdeg_ref[0][:, 0:1] + b2c_ref[...]
    u2 = aggu * rdeg_ref[1][:, 0:1] + b2u_ref[...]
    p_ref[...] = _dot(u2, wd1l_ref[...]) + bd1_ref[...]
    q_ref[...] = _dot(c2, wd1r_ref[...])


def _row_spec(bm, d):
    return pl.BlockSpec((bm, d), lambda i: (i, 0))


def _fix_spec(shape):
    nd = len(shape)
    return pl.BlockSpec(shape, lambda i, _nd=nd: (0,) * _nd)


def _t1_call(xu, xc, w1l, w1r, wr1l, wr1r, b1, br1):
    grid = N // BM
    return pl.pallas_call(
        _t1_body,
        grid=(grid,),
        in_specs=[_row_spec(BM, D), _row_spec(BM, D)]
        + [_fix_spec((D, D))] * 4 + [_fix_spec((1, D))] * 2,
        out_specs=[
            pl.BlockSpec((4, BM, W), lambda i: (0, i, 0)),
            _row_spec(BM, D), _row_spec(BM, D),
        ],
        out_shape=[
            jax.ShapeDtypeStruct((4, N, W), jnp.float32),
            jax.ShapeDtypeStruct((N, D), jnp.float32),
            jax.ShapeDtypeStruct((N, D), jnp.float32),
        ],
    )(xu, xc, w1l, w1r, wr1l, wr1r, b1, br1)


def _t2_call(agg, bc, bu, w2l, w2r, wr2l, wr2r, b2, br2):
    grid = N // BM
    return pl.pallas_call(
        _t2_body,
        grid=(grid,),
        in_specs=[
            pl.BlockSpec((2, 3, BM, W), lambda i: (0, 0, i, 0)),
            _row_spec(BM, D), _row_spec(BM, D),
        ] + [_fix_spec((D, D))] * 4 + [_fix_spec((1, D))] * 2,
        out_specs=[
            pl.BlockSpec((4, BM, W), lambda i: (0, i, 0)),
            _row_spec(BM, D), _row_spec(BM, D),
            pl.BlockSpec((2, BM, 16), lambda i: (0, i, 0)),
        ],
        out_shape=[
            jax.ShapeDtypeStruct((4, N, W), jnp.float32),
            jax.ShapeDtypeStruct((N, D), jnp.float32),
            jax.ShapeDtypeStruct((N, D), jnp.float32),
            jax.ShapeDtypeStruct((2, N, 16), jnp.float32),
        ],
    )(agg, bc, bu, w2l, w2r, wr2l, wr2r, b2, br2)


def _t3_call(agg, rdeg, b2c, b2u, wd1l, wd1r, bd1):
    grid = N // BM
    return pl.pallas_call(
        _t3_body,
        grid=(grid,),
        in_specs=[
            pl.BlockSpec((2, 2, BM, W), lambda i: (0, 0, i, 0)),
            pl.BlockSpec((2, BM, 16), lambda i: (0, i, 0)),
            _row_spec(BM, D), _row_spec(BM, D),
            _fix_spec((D, D)), _fix_spec((D, D)), _fix_spec((1, D)),
        ],
        out_specs=[_row_spec(BM, D), _row_spec(BM, D)],
        out_shape=[
            jax.ShapeDtypeStruct((N, D), jnp.float32),
            jax.ShapeDtypeStruct((N, D), jnp.float32),
        ],
    )(agg, rdeg, b2c, b2u, wd1l, wd1r, bd1)


def _seg_call(with_deg, g_flat, eidx, zeros_a, ones_a):
    npass = 3 if with_deg else 2
    mesh = plsc.VectorSubcoreMesh(core_axis_name="c", subcore_axis_name="s",
                                  num_cores=NC, num_subcores=NS)
    out_type = jax.ShapeDtypeStruct((2, npass, ACC_ROWS, W), jnp.float32)
    scratch = {
        "src_v": pltpu.VMEM((CH,), jnp.int32),
        "dst_v": pltpu.VMEM((CH,), jnp.int32),
        "rows_v": pltpu.VMEM((CH, W), jnp.float32),
        "acc": pltpu.VMEM_SHARED((ACC_ROWS, W), jnp.float32),
        "sem": pltpu.SemaphoreType.DMA,
    }

    def body(g_hbm, eidx_hbm, z_hbm, ones_hbm, agg_hbm, *,
             src_v, dst_v, rows_v, acc, sem):
        t = lax.axis_index("c")
        sid = lax.axis_index("s")
        r0 = sid * ZR

        def zero_acc():
            pltpu.sync_copy(z_hbm, rows_v)
            for j in range(NZB):
                pltpu.sync_copy(rows_v, acc.at[pl.ds(r0 + j * CH, CH)])

        zero_acc()
        plsc.subcore_barrier()
        for c in range(npass):
            if c == 2:
                pltpu.sync_copy(ones_hbm, rows_v)

                def dchunk(k, _):
                    e0 = sid * E_PER_TILE + k * CH
                    pltpu.sync_copy(eidx_hbm.at[t, 0, 1, pl.ds(e0, CH)],
                                    dst_v)
                    pltpu.sync_copy(rows_v, acc.at[dst_v], add=True)
                    return 0

                lax.fori_loop(0, N_CHUNKS, dchunk, 0)
            else:
                def chunk(k, _, _c=c):
                    e0 = sid * E_PER_TILE + k * CH
                    pltpu.sync_copy(eidx_hbm.at[t, _c, 0, pl.ds(e0, CH)],
                                    src_v)
                    pltpu.sync_copy(eidx_hbm.at[t, _c, 1, pl.ds(e0, CH)],
                                    dst_v)
                    pltpu.async_copy(g_hbm.at[src_v], rows_v, sem).wait()
                    pltpu.sync_copy(rows_v, acc.at[dst_v], add=True)
                    return 0

                lax.fori_loop(0, N_CHUNKS, chunk, 0)
            plsc.subcore_barrier()
            for j in range(NZB):
                pltpu.sync_copy(acc.at[pl.ds(r0 + j * CH, CH)], rows_v)
                pltpu.sync_copy(rows_v,
                                agg_hbm.at[t, c, pl.ds(r0 + j * CH, CH)])
            if c < npass - 1:
                zero_acc()
                plsc.subcore_barrier()

    fn = pl.kernel(body, out_type=out_type, mesh=mesh, scratch_types=scratch)
    return fn(g_flat, eidx, zeros_a, ones_a)


def _dec_call(p, q, lrow, lcol, wd2v, bdv):
    mesh = plsc.VectorSubcoreMesh(core_axis_name="c", subcore_axis_name="s",
                                  num_cores=NC, num_subcores=NS)
    scratch = {
        "ridx_v": pltpu.VMEM((LCH,), jnp.int32),
        "cidx_v": pltpu.VMEM((LCH,), jnp.int32),
        "prow_v": pltpu.VMEM((LCH, D), jnp.float32),
        "qrow_v": pltpu.VMEM((LCH, D), jnp.float32),
        "wd2_v": pltpu.VMEM((D,), jnp.float32),
        "bdv_v": pltpu.VMEM((16,), jnp.float32),
        "out_v": pltpu.VMEM((LCH,), jnp.float32),
        "sem": pltpu.SemaphoreType.DMA,
        "sem2": pltpu.SemaphoreType.DMA,
    }

    def body(p_hbm, q_hbm, lrow_hbm, lcol_hbm, wd2_hbm, bdv_hbm, out_hbm, *,
             ridx_v, cidx_v, prow_v, qrow_v, wd2_v, bdv_v, out_v, sem, sem2):
        cid = lax.axis_index("c")
        sid = lax.axis_index("s")
        wid = sid * NC + cid
        base = wid * L_PER_TILE
        pltpu.sync_copy(wd2_hbm, wd2_v)
        pltpu.sync_copy(bdv_hbm, bdv_v)
        lanes = lax.iota(jnp.int32, 16)

        def chunk(k, _):
            e0 = base + k * LCH
            pltpu.sync_copy(lrow_hbm.at[pl.ds(e0, LCH)], ridx_v)
            pltpu.sync_copy(lcol_hbm.at[pl.ds(e0, LCH)], cidx_v)
            pltpu.async_copy(p_hbm.at[ridx_v], prow_v, sem).wait()
            pltpu.async_copy(q_hbm.at[cidx_v], qrow_v, sem2).wait()

            def group(jg, _):
                svec = jnp.zeros((16,), jnp.float32)
                for j2 in range(16):
                    row = jg * 16 + j2
                    acc = bdv_v[...]
                    for kk in range(16):
                        pv = prow_v[row, pl.ds(kk * 16, 16)]
                        qv = qrow_v[row, pl.ds(kk * 16, 16)]
                        acc = acc + (jnp.maximum(pv + qv, 0.0)
                                     * wd2_v[pl.ds(kk * 16, 16)])
                    for sh in (8, 4, 2, 1):
                        acc = acc + _shuffle(acc,
                                             jnp.bitwise_xor(lanes, sh))
                    svec = jnp.where(lanes == j2, acc, svec)
                out_v[pl.ds(jg * 16, 16)] = svec
                return 0

            lax.fori_loop(0, LCH // 16, group, 0)
            pltpu.sync_copy(out_v, out_hbm.at[pl.ds(e0, LCH)])
            return 0

        lax.fori_loop(0, L_CHUNKS, chunk, 0)

    fn = pl.kernel(body, out_type=jax.ShapeDtypeStruct((NLP,), jnp.float32),
                   mesh=mesh, scratch_types=scratch)
    return fn(p, q, lrow, lcol, wd2v, bdv)


def kernel(x_user, x_course,
           W1_rates_l, b1_rates, W1_rates_r, W1_rev_l, b1_rev, W1_rev_r,
           W2_rates_l, b2_rates, W2_rates_r, W2_rev_l, b2_rev, W2_rev_r,
           Wd1, bd1, Wd2, bd2,
           ei_u2c, ei_c2u, edge_label_index):
    f32 = jnp.float32
    i32 = jnp.int32

    def prep(ei, t):
        src = jnp.pad(ei[0].astype(i32), (0, E_PAD - E))
        dst = jnp.pad(ei[1].astype(i32), (0, E_PAD - E), constant_values=TRASH)
        return jnp.stack([
            jnp.stack([src + (2 * t + 0) * N, dst]),
            jnp.stack([src + (2 * t + 1) * N, dst]),
        ])

    eidx = jnp.stack([prep(ei_u2c, 0), prep(ei_c2u, 1)])
    zeros_a = jnp.zeros((CH, W), f32)
    ones_a = jnp.ones((CH, W), f32)

    g1, bc, bu = _t1_call(x_user, x_course, W1_rates_l, W1_rates_r,
                          W1_rev_l, W1_rev_r, b1_rates.reshape(1, D),
                          b1_rev.reshape(1, D))
    agg1 = _seg_call(True, g1.reshape(4 * N, W), eidx, zeros_a, ones_a)
    g2, b2c, b2u, rdeg = _t2_call(agg1, bc, bu, W2_rates_l, W2_rates_r,
                                  W2_rev_l, W2_rev_r, b2_rates.reshape(1, D),
                                  b2_rev.reshape(1, D))
    agg2 = _seg_call(False, g2.reshape(4 * N, W), eidx, zeros_a, ones_a)
    p, q = _t3_call(agg2, rdeg, b2c, b2u, Wd1[:, :D], Wd1[:, D:],
                    bd1.reshape(1, D))

    lrow = jnp.pad(edge_label_index[0].astype(i32), (0, NLP - NL))
    lcol = jnp.pad(edge_label_index[1].astype(i32), (0, NLP - NL))
    wd2v = Wd2.reshape(D)
    bdv = jnp.full((16,), bd2[0] / 16.0, f32)
    outp = _dec_call(p, q, lrow, lcol, wd2v, bdv)
    return outp[:NL]


# speedup vs baseline: 1.9808x; 1.2934x over previous
import functools

import jax
import jax.numpy as jnp
from jax import lax
from jax.experimental import pallas as pl
from jax.experimental.pallas import tpu as pltpu
from jax.experimental.pallas import tpu_sc as plsc

N = 10000
D = 256
E = 160000
NL = 50000

NC, NS = 2, 16
E_PER_TILE = 10240
E_PAD = NS * E_PER_TILE
CH = 128
N_CHUNKS = E_PER_TILE // CH
ACC_ROWS = 10240
ZR = ACC_ROWS // NS
NZB = ZR // CH
TRASH = N
W = 128

NLP = 53248
L_PER_TILE = NLP // (NC * NS)
LCH = 128
L_CHUNKS = L_PER_TILE // LCH

BM = 2000
_DN = (((1,), (1,)), ((), ()))


def _dot(x, w):
    return lax.dot_general(x, w, _DN, preferred_element_type=jnp.float32)


_GDN = lax.GatherDimensionNumbers(offset_dims=(), collapsed_slice_dims=(0,),
                                  start_index_map=(0,))


def _shuffle(v, idx):
    return lax.gather(v, idx[:, None], _GDN, (1,),
                      mode=lax.GatherScatterMode.PROMISE_IN_BOUNDS)


def _t1_body(xu_ref, xc_ref, w1l_ref, w1r_ref, wr1l_ref, wr1r_ref,
             b1_ref, br1_ref, g_ref, bc_ref, bu_ref):
    xu = xu_ref[...]
    xc = xc_ref[...]
    g1u = _dot(xu, w1l_ref[...])
    g1c = _dot(xc, wr1l_ref[...])
    g_ref[0] = g1u[:, :128]
    g_ref[1] = g1u[:, 128:]
    g_ref[2] = g1c[:, :128]
    g_ref[3] = g1c[:, 128:]
    bc_ref[...] = _dot(xc, w1r_ref[...]) + b1_ref[...]
    bu_ref[...] = _dot(xu, wr1r_ref[...]) + br1_ref[...]


def _t2_body(agg_ref, bc_ref, bu_ref, w2l_ref, w2r_ref,
             wr2l_ref, wr2r_ref, b2_ref, br2_ref,
             g2_ref, b2c_ref, b2u_ref, rdeg_ref):
    aggc = jnp.concatenate([agg_ref[0, 0], agg_ref[0, 1]], axis=1)
    aggu = jnp.concatenate([agg_ref[1, 0], agg_ref[1, 1]], axis=1)
    rdegc = 1.0 / jnp.maximum(agg_ref[0, 2][:, 0:1], 1.0)
    rdegu = 1.0 / jnp.maximum(agg_ref[1, 2][:, 0:1], 1.0)
    c1 = jnp.maximum(aggc * rdegc + bc_ref[...], 0.0)
    u1 = jnp.maximum(aggu * rdegu + bu_ref[...], 0.0)
    g2u = _dot(u1, w2l_ref[...])
    g2c = _dot(c1, wr2l_ref[...])
    g2_ref[0] = g2u[:, :128]
    g2_ref[1] = g2u[:, 128:]
    g2_ref[2] = g2c[:, :128]
    g2_ref[3] = g2c[:, 128:]
    b2c_ref[...] = _dot(c1, w2r_ref[...]) + b2_ref[...]
    b2u_ref[...] = _dot(u1, wr2r_ref[...]) + br2_ref[...]
    bm = rdegc.shape[0]
    rdeg_ref[0] = jnp.broadcast_to(rdegc, (bm, 16))
    rdeg_ref[1] = jnp.broadcast_to(rdegu, (bm, 16))


def _t3_body(agg_ref, rdeg_ref, b2c_ref, b2u_ref, wd1l_ref, wd1r_ref,
             bd1_ref, p_ref, q_ref):
    aggc = jnp.concatenate([agg_ref[0, 0], agg_ref[0, 1]], axis=1)
    aggu = jnp.concatenate([agg_ref[1, 0], agg_ref[1, 1]], axis=1)
    c2 = aggc * rdeg_ref[0][:, 0:1] + b2c_ref[...]
    u2 = aggu * rdeg_ref[1][:, 0:1] + b2u_ref[...]
    p_ref[...] = _dot(u2, wd1l_ref[...]) + bd1_ref[...]
    q_ref[...] = _dot(c2, wd1r_ref[...])


def _row_spec(bm, d):
    return pl.BlockSpec((bm, d), lambda i: (i, 0))


def _fix_spec(shape):
    nd = len(shape)
    return pl.BlockSpec(shape, lambda i, _nd=nd: (0,) * _nd)


def _t1_call(xu, xc, w1l, w1r, wr1l, wr1r, b1, br1):
    grid = N // BM
    return pl.pallas_call(
        _t1_body,
        grid=(grid,),
        in_specs=[_row_spec(BM, D), _row_spec(BM, D)]
        + [_fix_spec((D, D))] * 4 + [_fix_spec((1, D))] * 2,
        out_specs=[
            pl.BlockSpec((4, BM, W), lambda i: (0, i, 0)),
            _row_spec(BM, D), _row_spec(BM, D),
        ],
        out_shape=[
            jax.ShapeDtypeStruct((4, N, W), jnp.float32),
            jax.ShapeDtypeStruct((N, D), jnp.float32),
            jax.ShapeDtypeStruct((N, D), jnp.float32),
        ],
    )(xu, xc, w1l, w1r, wr1l, wr1r, b1, br1)


def _t2_call(agg, bc, bu, w2l, w2r, wr2l, wr2r, b2, br2):
    grid = N // BM
    return pl.pallas_call(
        _t2_body,
        grid=(grid,),
        in_specs=[
            pl.BlockSpec((2, 3, BM, W), lambda i: (0, 0, i, 0)),
            _row_spec(BM, D), _row_spec(BM, D),
        ] + [_fix_spec((D, D))] * 4 + [_fix_spec((1, D))] * 2,
        out_specs=[
            pl.BlockSpec((4, BM, W), lambda i: (0, i, 0)),
            _row_spec(BM, D), _row_spec(BM, D),
            pl.BlockSpec((2, BM, 16), lambda i: (0, i, 0)),
        ],
        out_shape=[
            jax.ShapeDtypeStruct((4, N, W), jnp.float32),
            jax.ShapeDtypeStruct((N, D), jnp.float32),
            jax.ShapeDtypeStruct((N, D), jnp.float32),
            jax.ShapeDtypeStruct((2, N, 16), jnp.float32),
        ],
    )(agg, bc, bu, w2l, w2r, wr2l, wr2r, b2, br2)


def _t3_call(agg, rdeg, b2c, b2u, wd1l, wd1r, bd1):
    grid = N // BM
    return pl.pallas_call(
        _t3_body,
        grid=(grid,),
        in_specs=[
            pl.BlockSpec((2, 2, BM, W), lambda i: (0, 0, i, 0)),
            pl.BlockSpec((2, BM, 16), lambda i: (0, i, 0)),
            _row_spec(BM, D), _row_spec(BM, D),
            _fix_spec((D, D)), _fix_spec((D, D)), _fix_spec((1, D)),
        ],
        out_specs=[_row_spec(BM, D), _row_spec(BM, D)],
        out_shape=[
            jax.ShapeDtypeStruct((N, D), jnp.float32),
            jax.ShapeDtypeStruct((N, D), jnp.float32),
        ],
    )(agg, rdeg, b2c, b2u, wd1l, wd1r, bd1)


def _seg_call(with_deg, g_flat, eidx, zeros_a, ones_a):
    npass = 3 if with_deg else 2
    mesh = plsc.VectorSubcoreMesh(core_axis_name="c", subcore_axis_name="s",
                                  num_cores=NC, num_subcores=NS)
    out_type = jax.ShapeDtypeStruct((2, npass, ACC_ROWS, W), jnp.float32)
    scratch = {
        "idx_v": pltpu.VMEM((2, 2, CH), jnp.int32),
        "rows_v": pltpu.VMEM((2, CH, W), jnp.float32),
        "acc": pltpu.VMEM_SHARED((ACC_ROWS, W), jnp.float32),
        "sems": pltpu.SemaphoreType.DMA((2,)),
    }

    def body(g_hbm, eidx_hbm, z_hbm, ones_hbm, agg_hbm, *,
             idx_v, rows_v, acc, sems):
        t = lax.axis_index("c")
        sid = lax.axis_index("s")
        r0 = sid * ZR

        def zero_acc():
            pltpu.sync_copy(z_hbm, rows_v.at[0])
            for j in range(NZB):
                pltpu.sync_copy(rows_v.at[0], acc.at[pl.ds(r0 + j * CH, CH)])

        zero_acc()
        plsc.subcore_barrier()
        for c in range(npass):
            if c == 2:
                pltpu.sync_copy(ones_hbm, rows_v.at[0])

                def dchunk(k, _):
                    pltpu.sync_copy(eidx_hbm.at[t, 0, sid, k], idx_v.at[0])
                    pltpu.sync_copy(rows_v.at[0], acc.at[idx_v.at[0, 1]],
                                    add=True)
                    return 0

                lax.fori_loop(0, N_CHUNKS, dchunk, 0)
            else:
                for b in range(2):
                    pltpu.sync_copy(eidx_hbm.at[t, c, sid, b], idx_v.at[b])
                    pltpu.async_copy(g_hbm.at[idx_v.at[b, 0]], rows_v.at[b],
                                     sems.at[b])

                def chunk(k, _, _c=c):
                    cb = lax.rem(k, 2)
                    pltpu.make_async_copy(g_hbm.at[idx_v.at[cb, 0]],
                                          rows_v.at[cb],
                                          sems.at[cb]).wait()
                    pltpu.sync_copy(rows_v.at[cb], acc.at[idx_v.at[cb, 1]],
                                    add=True)

                    @pl.when(k + 2 < N_CHUNKS)
                    def _():
                        pltpu.sync_copy(eidx_hbm.at[t, _c, sid, k + 2],
                                        idx_v.at[cb])
                        pltpu.async_copy(g_hbm.at[idx_v.at[cb, 0]],
                                         rows_v.at[cb], sems.at[cb])

                    return 0

                lax.fori_loop(0, N_CHUNKS, chunk, 0)
            plsc.subcore_barrier()
            for j in range(NZB):
                pltpu.sync_copy(acc.at[pl.ds(r0 + j * CH, CH)], rows_v.at[0])
                pltpu.sync_copy(rows_v.at[0],
                                agg_hbm.at[t, c, pl.ds(r0 + j * CH, CH)])
            if c < npass - 1:
                zero_acc()
                plsc.subcore_barrier()

    fn = pl.kernel(body, out_type=out_type, mesh=mesh, scratch_types=scratch)
    return fn(g_flat, eidx, zeros_a, ones_a)


def _dec_call(p, q, lrow, lcol, wd2v, bdv):
    mesh = plsc.VectorSubcoreMesh(core_axis_name="c", subcore_axis_name="s",
                                  num_cores=NC, num_subcores=NS)
    scratch = {
        "ridx_v": pltpu.VMEM((L_CHUNKS, LCH), jnp.int32),
        "cidx_v": pltpu.VMEM((L_CHUNKS, LCH), jnp.int32),
        "prow_v": pltpu.VMEM((LCH, D), jnp.float32),
        "qrow_v": pltpu.VMEM((LCH, D), jnp.float32),
        "wd2_v": pltpu.VMEM((D,), jnp.float32),
        "bdv_v": pltpu.VMEM((16,), jnp.float32),
        "out_v": pltpu.VMEM((L_PER_TILE,), jnp.float32),
        "sem": pltpu.SemaphoreType.DMA,
        "sem2": pltpu.SemaphoreType.DMA,
    }

    def body(p_hbm, q_hbm, lrow_hbm, lcol_hbm, wd2_hbm, bdv_hbm, out_hbm, *,
             ridx_v, cidx_v, prow_v, qrow_v, wd2_v, bdv_v, out_v, sem, sem2):
        cid = lax.axis_index("c")
        sid = lax.axis_index("s")
        wid = sid * NC + cid
        base = wid * L_PER_TILE
        pltpu.sync_copy(wd2_hbm, wd2_v)
        pltpu.sync_copy(bdv_hbm, bdv_v)
        pltpu.sync_copy(lrow_hbm.at[wid], ridx_v)
        pltpu.sync_copy(lcol_hbm.at[wid], cidx_v)
        lanes = lax.iota(jnp.int32, 16)

        def chunk(k, _):
            pltpu.async_copy(p_hbm.at[ridx_v.at[k]], prow_v, sem)
            pltpu.async_copy(q_hbm.at[cidx_v.at[k]], qrow_v, sem2)
            pltpu.make_async_copy(p_hbm.at[ridx_v.at[k]], prow_v, sem).wait()
            pltpu.make_async_copy(q_hbm.at[cidx_v.at[k]], qrow_v,
                                  sem2).wait()

            def group(jg, _):
                svec = jnp.zeros((16,), jnp.float32)
                for j2 in range(16):
                    row = jg * 16 + j2
                    acc = bdv_v[...]
                    for kk in range(16):
                        pv = prow_v[row, pl.ds(kk * 16, 16)]
                        qv = qrow_v[row, pl.ds(kk * 16, 16)]
                        acc = acc + (jnp.maximum(pv + qv, 0.0)
                                     * wd2_v[pl.ds(kk * 16, 16)])
                    for sh in (8, 4, 2, 1):
                        acc = acc + _shuffle(acc,
                                             jnp.bitwise_xor(lanes, sh))
                    svec = jnp.where(lanes == j2, acc, svec)
                out_v[pl.ds(k * LCH + jg * 16, 16)] = svec
                return 0

            lax.fori_loop(0, LCH // 16, group, 0)
            return 0

        lax.fori_loop(0, L_CHUNKS, chunk, 0)
        pltpu.sync_copy(out_v, out_hbm.at[pl.ds(base, L_PER_TILE)])

    fn = pl.kernel(body, out_type=jax.ShapeDtypeStruct((NLP,), jnp.float32),
                   mesh=mesh, scratch_types=scratch)
    return fn(p, q, lrow, lcol, wd2v, bdv)


def kernel(x_user, x_course,
           W1_rates_l, b1_rates, W1_rates_r, W1_rev_l, b1_rev, W1_rev_r,
           W2_rates_l, b2_rates, W2_rates_r, W2_rev_l, b2_rev, W2_rev_r,
           Wd1, bd1, Wd2, bd2,
           ei_u2c, ei_c2u, edge_label_index):
    f32 = jnp.float32
    i32 = jnp.int32

    def prep(ei, t):
        src_ = jnp.pad(ei[0].astype(i32), (0, E_PAD - E))
        dst = jnp.pad(ei[1].astype(i32), (0, E_PAD - E),
                      constant_values=TRASH)
        dst = dst.reshape(NS, N_CHUNKS, 1, CH)
        out = []
        for c in range(2):
            s = (src_ + (2 * t + c) * N).reshape(NS, N_CHUNKS, 1, CH)
            out.append(jnp.concatenate([s, dst], axis=2))
        return jnp.stack(out)

    eidx = jnp.stack([prep(ei_u2c, 0), prep(ei_c2u, 1)])
    zeros_a = jnp.zeros((CH, W), f32)
    ones_a = jnp.ones((CH, W), f32)

    g1, bc, bu = _t1_call(x_user, x_course, W1_rates_l, W1_rates_r,
                          W1_rev_l, W1_rev_r, b1_rates.reshape(1, D),
                          b1_rev.reshape(1, D))
    agg1 = _seg_call(True, g1.reshape(4 * N, W), eidx, zeros_a, ones_a)
    g2, b2c, b2u, rdeg = _t2_call(agg1, bc, bu, W2_rates_l, W2_rates_r,
                                  W2_rev_l, W2_rev_r, b2_rates.reshape(1, D),
                                  b2_rev.reshape(1, D))
    agg2 = _seg_call(False, g2.reshape(4 * N, W), eidx, zeros_a, ones_a)
    p, q = _t3_call(agg2, rdeg, b2c, b2u, Wd1[:, :D], Wd1[:, D:],
                    bd1.reshape(1, D))

    lrow = jnp.pad(edge_label_index[0].astype(i32),
                   (0, NLP - NL)).reshape(NC * NS, L_CHUNKS, LCH)
    lcol = jnp.pad(edge_label_index[1].astype(i32),
                   (0, NLP - NL)).reshape(NC * NS, L_CHUNKS, LCH)
    wd2v = Wd2.reshape(D)
    bdv = jnp.full((16,), bd2[0] / 16.0, f32)
    outp = _dec_call(p, q, lrow, lcol, wd2v, bdv)
    return outp[:NL]


# speedup vs baseline: 2.1235x; 1.0720x over previous
import functools

import jax
import jax.numpy as jnp
from jax import lax
from jax.experimental import pallas as pl
from jax.experimental.pallas import tpu as pltpu
from jax.experimental.pallas import tpu_sc as plsc

N = 10000
D = 256
E = 160000
NL = 50000

NC, NS = 2, 16
E_PER_TILE = 10240
E_PAD = NS * E_PER_TILE
CH = 64
N_CHUNKS = E_PER_TILE // CH
ACC_ROWS = 10240
ZR = ACC_ROWS // NS
ZB = 64
NZB = ZR // ZB
TRASH = N
W = 128

NLP = 53248
L_PER_TILE = NLP // (NC * NS)
LCH = 64
L_CHUNKS = L_PER_TILE // LCH

BM = 2000
_DN = (((1,), (1,)), ((), ()))


def _dot(x, w):
    return lax.dot_general(x, w, _DN, preferred_element_type=jnp.float32)


_GDN = lax.GatherDimensionNumbers(offset_dims=(), collapsed_slice_dims=(0,),
                                  start_index_map=(0,))


def _shuffle(v, idx):
    return lax.gather(v, idx[:, None], _GDN, (1,),
                      mode=lax.GatherScatterMode.PROMISE_IN_BOUNDS)


def _t1_body(xu_ref, xc_ref, w1l_ref, w1r_ref, wr1l_ref, wr1r_ref,
             b1_ref, br1_ref, g_ref, bc_ref, bu_ref):
    xu = xu_ref[...]
    xc = xc_ref[...]
    g1u = _dot(xu, w1l_ref[...])
    g1c = _dot(xc, wr1l_ref[...])
    g_ref[0] = g1u[:, :128]
    g_ref[1] = g1u[:, 128:]
    g_ref[2] = g1c[:, :128]
    g_ref[3] = g1c[:, 128:]
    bc_ref[...] = _dot(xc, w1r_ref[...]) + b1_ref[...]
    bu_ref[...] = _dot(xu, wr1r_ref[...]) + br1_ref[...]


def _t2_body(agg_ref, bc_ref, bu_ref, w2l_ref, w2r_ref,
             wr2l_ref, wr2r_ref, b2_ref, br2_ref,
             g2_ref, b2c_ref, b2u_ref, rdeg_ref):
    aggc = jnp.concatenate([agg_ref[0, 0], agg_ref[0, 1]], axis=1)
    aggu = jnp.concatenate([agg_ref[1, 0], agg_ref[1, 1]], axis=1)
    rdegc = 1.0 / jnp.maximum(agg_ref[0, 2][:, 0:1], 1.0)
    rdegu = 1.0 / jnp.maximum(agg_ref[1, 2][:, 0:1], 1.0)
    c1 = jnp.maximum(aggc * rdegc + bc_ref[...], 0.0)
    u1 = jnp.maximum(aggu * rdegu + bu_ref[...], 0.0)
    g2u = _dot(u1, w2l_ref[...])
    g2c = _dot(c1, wr2l_ref[...])
    g2_ref[0] = g2u[:, :128]
    g2_ref[1] = g2u[:, 128:]
    g2_ref[2] = g2c[:, :128]
    g2_ref[3] = g2c[:, 128:]
    b2c_ref[...] = _dot(c1, w2r_ref[...]) + b2_ref[...]
    b2u_ref[...] = _dot(u1, wr2r_ref[...]) + br2_ref[...]
    bm = rdegc.shape[0]
    rdeg_ref[0] = jnp.broadcast_to(rdegc, (bm, 16))
    rdeg_ref[1] = jnp.broadcast_to(rdegu, (bm, 16))


def _t3_body(agg_ref, rdeg_ref, b2c_ref, b2u_ref, wd1l_ref, wd1r_ref,
             bd1_ref, p_ref, q_ref):
    aggc = jnp.concatenate([agg_ref[0, 0], agg_ref[0, 1]], axis=1)
    aggu = jnp.concatenate([agg_ref[1, 0], agg_ref[1, 1]], axis=1)
    c2 = aggc * rdeg_ref[0][:, 0:1] + b2c_ref[...]
    u2 = aggu * rdeg_ref[1][:, 0:1] + b2u_ref[...]
    p_ref[...] = _dot(u2, wd1l_ref[...]) + bd1_ref[...]
    q_ref[...] = _dot(c2, wd1r_ref[...])


def _row_spec(bm, d):
    return pl.BlockSpec((bm, d), lambda i: (i, 0))


def _fix_spec(shape):
    nd = len(shape)
    return pl.BlockSpec(shape, lambda i, _nd=nd: (0,) * _nd)


def _t1_call(xu, xc, w1l, w1r, wr1l, wr1r, b1, br1):
    grid = N // BM
    return pl.pallas_call(
        _t1_body,
        grid=(grid,),
        in_specs=[_row_spec(BM, D), _row_spec(BM, D)]
        + [_fix_spec((D, D))] * 4 + [_fix_spec((1, D))] * 2,
        out_specs=[
            pl.BlockSpec((4, BM, W), lambda i: (0, i, 0)),
            _row_spec(BM, D), _row_spec(BM, D),
        ],
        out_shape=[
            jax.ShapeDtypeStruct((4, N, W), jnp.float32),
            jax.ShapeDtypeStruct((N, D), jnp.float32),
            jax.ShapeDtypeStruct((N, D), jnp.float32),
        ],
    )(xu, xc, w1l, w1r, wr1l, wr1r, b1, br1)


def _t2_call(agg, bc, bu, w2l, w2r, wr2l, wr2r, b2, br2):
    grid = N // BM
    return pl.pallas_call(
        _t2_body,
        grid=(grid,),
        in_specs=[
            pl.BlockSpec((2, 3, BM, W), lambda i: (0, 0, i, 0)),
            _row_spec(BM, D), _row_spec(BM, D),
        ] + [_fix_spec((D, D))] * 4 + [_fix_spec((1, D))] * 2,
        out_specs=[
            pl.BlockSpec((4, BM, W), lambda i: (0, i, 0)),
            _row_spec(BM, D), _row_spec(BM, D),
            pl.BlockSpec((2, BM, 16), lambda i: (0, i, 0)),
        ],
        out_shape=[
            jax.ShapeDtypeStruct((4, N, W), jnp.float32),
            jax.ShapeDtypeStruct((N, D), jnp.float32),
            jax.ShapeDtypeStruct((N, D), jnp.float32),
            jax.ShapeDtypeStruct((2, N, 16), jnp.float32),
        ],
    )(agg, bc, bu, w2l, w2r, wr2l, wr2r, b2, br2)


def _t3_call(agg, rdeg, b2c, b2u, wd1l, wd1r, bd1):
    grid = N // BM
    return pl.pallas_call(
        _t3_body,
        grid=(grid,),
        in_specs=[
            pl.BlockSpec((2, 2, BM, W), lambda i: (0, 0, i, 0)),
            pl.BlockSpec((2, BM, 16), lambda i: (0, i, 0)),
            _row_spec(BM, D), _row_spec(BM, D),
            _fix_spec((D, D)), _fix_spec((D, D)), _fix_spec((1, D)),
        ],
        out_specs=[_row_spec(BM, D), _row_spec(BM, D)],
        out_shape=[
            jax.ShapeDtypeStruct((N, D), jnp.float32),
            jax.ShapeDtypeStruct((N, D), jnp.float32),
        ],
    )(agg, rdeg, b2c, b2u, wd1l, wd1r, bd1)


def _seg_call(with_deg, g_flat, eidx, zeros_a, ones_a):
    npass = 3 if with_deg else 2
    mesh = plsc.VectorSubcoreMesh(core_axis_name="c", subcore_axis_name="s",
                                  num_cores=NC, num_subcores=NS)
    out_type = jax.ShapeDtypeStruct((2, npass, ACC_ROWS, W), jnp.float32)
    scratch = {
        "idx_v": pltpu.VMEM((8, 2, CH), jnp.int32),
        "rows_v": pltpu.VMEM((4, CH, W), jnp.float32),
        "acc": pltpu.VMEM_SHARED((ACC_ROWS, W), jnp.float32),
        "gsems": pltpu.SemaphoreType.DMA((4,)),
        "ssems": pltpu.SemaphoreType.DMA((4,)),
    }

    def body(g_hbm, eidx_hbm, z_hbm, ones_hbm, agg_hbm, *,
             idx_v, rows_v, acc, gsems, ssems):
        t = lax.axis_index("c")
        sid = lax.axis_index("s")
        r0 = sid * ZR

        def zero_acc():
            pltpu.sync_copy(z_hbm, rows_v.at[0])
            for j in range(NZB):
                pltpu.sync_copy(rows_v.at[0],
                                acc.at[pl.ds(r0 + j * ZB, ZB)])

        zero_acc()
        plsc.subcore_barrier()
        for c in range(npass):
            if c == 2:
                pltpu.sync_copy(ones_hbm, rows_v.at[0])

                def dchunk(k, _):
                    pltpu.sync_copy(eidx_hbm.at[t, 0, sid, k], idx_v.at[0])
                    pltpu.sync_copy(rows_v.at[0], acc.at[idx_v.at[0, 1]],
                                    add=True)
                    return 0

                lax.fori_loop(0, N_CHUNKS, dchunk, 0)
            else:
                for b in range(4):
                    pltpu.sync_copy(eidx_hbm.at[t, c, sid, b], idx_v.at[b])
                    pltpu.async_copy(g_hbm.at[idx_v.at[b, 0]], rows_v.at[b],
                                     gsems.at[b])

                def chunk(k, _, _c=c):
                    cb = lax.rem(k, 4)
                    ib = lax.rem(k, 8)
                    pltpu.make_async_copy(g_hbm.at[idx_v.at[ib, 0]],
                                          rows_v.at[cb],
                                          gsems.at[cb]).wait()
                    pltpu.async_copy(rows_v.at[cb], acc.at[idx_v.at[ib, 1]],
                                     ssems.at[cb], add=True)

                    @pl.when(k + 4 < N_CHUNKS)
                    def _():
                        nib = lax.rem(k + 4, 8)
                        pltpu.sync_copy(eidx_hbm.at[t, _c, sid, k + 4],
                                        idx_v.at[nib])
                        pltpu.make_async_copy(rows_v.at[cb],
                                              acc.at[idx_v.at[ib, 1]],
                                              ssems.at[cb]).wait()
                        pltpu.async_copy(g_hbm.at[idx_v.at[nib, 0]],
                                         rows_v.at[cb], gsems.at[cb])

                    @pl.when(k + 4 >= N_CHUNKS)
                    def _():
                        pltpu.make_async_copy(rows_v.at[cb],
                                              acc.at[idx_v.at[ib, 1]],
                                              ssems.at[cb]).wait()

                    return 0

                lax.fori_loop(0, N_CHUNKS, chunk, 0)
            plsc.subcore_barrier()
            for j in range(NZB):
                pltpu.sync_copy(acc.at[pl.ds(r0 + j * ZB, ZB)],
                                rows_v.at[0])
                pltpu.sync_copy(rows_v.at[0],
                                agg_hbm.at[t, c, pl.ds(r0 + j * ZB, ZB)])
            if c < npass - 1:
                zero_acc()
                plsc.subcore_barrier()

    fn = pl.kernel(body, out_type=out_type, mesh=mesh, scratch_types=scratch)
    return fn(g_flat, eidx, zeros_a, ones_a)


def _dec_call(p, q, lrow, lcol, wd2v, bdv):
    mesh = plsc.VectorSubcoreMesh(core_axis_name="c", subcore_axis_name="s",
                                  num_cores=NC, num_subcores=NS)
    scratch = {
        "ridx_v": pltpu.VMEM((L_CHUNKS, LCH), jnp.int32),
        "cidx_v": pltpu.VMEM((L_CHUNKS, LCH), jnp.int32),
        "prow_v": pltpu.VMEM((2, LCH, D), jnp.float32),
        "qrow_v": pltpu.VMEM((2, LCH, D), jnp.float32),
        "wd2_v": pltpu.VMEM((D,), jnp.float32),
        "bdv_v": pltpu.VMEM((16,), jnp.float32),
        "out_v": pltpu.VMEM((L_PER_TILE,), jnp.float32),
        "psems": pltpu.SemaphoreType.DMA((2,)),
        "qsems": pltpu.SemaphoreType.DMA((2,)),
    }

    def body(p_hbm, q_hbm, lrow_hbm, lcol_hbm, wd2_hbm, bdv_hbm, out_hbm, *,
             ridx_v, cidx_v, prow_v, qrow_v, wd2_v, bdv_v, out_v,
             psems, qsems):
        cid = lax.axis_index("c")
        sid = lax.axis_index("s")
        wid = sid * NC + cid
        base = wid * L_PER_TILE
        pltpu.sync_copy(wd2_hbm, wd2_v)
        pltpu.sync_copy(bdv_hbm, bdv_v)
        pltpu.sync_copy(lrow_hbm.at[wid], ridx_v)
        pltpu.sync_copy(lcol_hbm.at[wid], cidx_v)
        lanes = lax.iota(jnp.int32, 16)
        for b in range(2):
            pltpu.async_copy(p_hbm.at[ridx_v.at[b]], prow_v.at[b],
                             psems.at[b])
            pltpu.async_copy(q_hbm.at[cidx_v.at[b]], qrow_v.at[b],
                             qsems.at[b])

        def chunk(k, _):
            cb = lax.rem(k, 2)
            pltpu.make_async_copy(p_hbm.at[ridx_v.at[k]], prow_v.at[cb],
                                  psems.at[cb]).wait()
            pltpu.make_async_copy(q_hbm.at[cidx_v.at[k]], qrow_v.at[cb],
                                  qsems.at[cb]).wait()

            def group(jg, _):
                svec = jnp.zeros((16,), jnp.float32)
                for j2 in range(16):
                    row = jg * 16 + j2
                    acc = bdv_v[...]
                    for kk in range(16):
                        pv = prow_v[cb, row, pl.ds(kk * 16, 16)]
                        qv = qrow_v[cb, row, pl.ds(kk * 16, 16)]
                        acc = acc + (jnp.maximum(pv + qv, 0.0)
                                     * wd2_v[pl.ds(kk * 16, 16)])
                    for sh in (8, 4, 2, 1):
                        acc = acc + _shuffle(acc,
                                             jnp.bitwise_xor(lanes, sh))
                    svec = jnp.where(lanes == j2, acc, svec)
                out_v[pl.ds(k * LCH + jg * 16, 16)] = svec
                return 0

            lax.fori_loop(0, LCH // 16, group, 0)

            @pl.when(k + 2 < L_CHUNKS)
            def _():
                pltpu.async_copy(p_hbm.at[ridx_v.at[k + 2]], prow_v.at[cb],
                                 psems.at[cb])
                pltpu.async_copy(q_hbm.at[cidx_v.at[k + 2]], qrow_v.at[cb],
                                 qsems.at[cb])

            return 0

        lax.fori_loop(0, L_CHUNKS, chunk, 0)
        pltpu.sync_copy(out_v, out_hbm.at[pl.ds(base, L_PER_TILE)])

    fn = pl.kernel(body, out_type=jax.ShapeDtypeStruct((NLP,), jnp.float32),
                   mesh=mesh, scratch_types=scratch)
    return fn(p, q, lrow, lcol, wd2v, bdv)


def kernel(x_user, x_course,
           W1_rates_l, b1_rates, W1_rates_r, W1_rev_l, b1_rev, W1_rev_r,
           W2_rates_l, b2_rates, W2_rates_r, W2_rev_l, b2_rev, W2_rev_r,
           Wd1, bd1, Wd2, bd2,
           ei_u2c, ei_c2u, edge_label_index):
    f32 = jnp.float32
    i32 = jnp.int32

    def prep(ei, t):
        src_ = jnp.pad(ei[0].astype(i32), (0, E_PAD - E))
        dst = jnp.pad(ei[1].astype(i32), (0, E_PAD - E),
                      constant_values=TRASH)
        dst = dst.reshape(NS, N_CHUNKS, 1, CH)
        out = []
        for c in range(2):
            s = (src_ + (2 * t + c) * N).reshape(NS, N_CHUNKS, 1, CH)
            out.append(jnp.concatenate([s, dst], axis=2))
        return jnp.stack(out)

    eidx = jnp.stack([prep(ei_u2c, 0), prep(ei_c2u, 1)])
    zeros_a = jnp.zeros((ZB, W), f32)
    ones_a = jnp.ones((CH, W), f32)

    g1, bc, bu = _t1_call(x_user, x_course, W1_rates_l, W1_rates_r,
                          W1_rev_l, W1_rev_r, b1_rates.reshape(1, D),
                          b1_rev.reshape(1, D))
    agg1 = _seg_call(True, g1.reshape(4 * N, W), eidx, zeros_a, ones_a)
    g2, b2c, b2u, rdeg = _t2_call(agg1, bc, bu, W2_rates_l, W2_rates_r,
                                  W2_rev_l, W2_rev_r, b2_rates.reshape(1, D),
                                  b2_rev.reshape(1, D))
    agg2 = _seg_call(False, g2.reshape(4 * N, W), eidx, zeros_a, ones_a)
    p, q = _t3_call(agg2, rdeg, b2c, b2u, Wd1[:, :D], Wd1[:, D:],
                    bd1.reshape(1, D))

    lrow = jnp.pad(edge_label_index[0].astype(i32),
                   (0, NLP - NL)).reshape(NC * NS, L_CHUNKS, LCH)
    lcol = jnp.pad(edge_label_index[1].astype(i32),
                   (0, NLP - NL)).reshape(NC * NS, L_CHUNKS, LCH)
    wd2v = Wd2.reshape(D)
    bdv = jnp.full((16,), bd2[0] / 16.0, f32)
    outp = _dec_call(p, q, lrow, lcol, wd2v, bdv)
    return outp[:NL]
